# baseline (speedup 1.0000x reference)
import math
import functools

import jax
import jax.numpy as jnp
from jax import lax
from jax.experimental import pallas as pl
from jax.experimental.pallas import tpu as pltpu
from jax.experimental.pallas import tpu_sc as plsc

SIZE_ = 32000
PAD_ = 0
SMOOTH_ = 0.1
CONF_ = 1.0 - SMOOTH_
SVAL_ = SMOOTH_ / (SIZE_ - 2)
K_ = (SIZE_ - 2) * SVAL_ * math.log(SVAL_) + CONF_ * math.log(CONF_)

B_ = 2048
RTC_ = 1536
BC_ = 1280

_NC = 2
_NS = 16
_NW = _NC * _NS
_RPW = (B_ - RTC_) // _NW
_L = 16
_CR = 8
_CC = 6400
_NRC = _RPW // _CR
_NCC = SIZE_ // _CC


def _tc_body(out_ref, t_ref, acc_ref):
    j = pl.program_id(0)
    out_blk = out_ref[...]
    t_blk = t_ref[...]
    nonpad = t_blk != PAD_

    colids = j * BC_ + lax.broadcasted_iota(jnp.int32, (RTC_, BC_), 1)
    w = jnp.where(colids == t_blk, CONF_, SVAL_)
    w = jnp.where(colids == 0, 0.0, w)
    w = jnp.where(nonpad, w, 0.0)
    partial = jnp.sum(out_blk * w)

    @pl.when(j == 0)
    def _init():
        cnt = jnp.sum(nonpad.astype(jnp.float32))
        acc_ref[0, 0] = K_ * cnt

    acc_ref[0, 0] = acc_ref[0, 0] - partial


def _sc_body(out_hbm, tgt_hbm, part_hbm, tgt_v, buf0, buf1, part_v,
             sem0, sem1):
    wid = lax.axis_index("s") * _NC + lax.axis_index("c")
    base = RTC_ + wid * _RPW
    pltpu.sync_copy(tgt_hbm.at[pl.ds(base, _RPW)], tgt_v)

    lanes = lax.iota(jnp.int32, _L)
    zeros = jnp.zeros((_L,), jnp.float32)

    bufs = (buf0, buf1)
    sems = (sem0, sem1)
    chunks = [(rc, cc) for rc in range(_NRC) for cc in range(_NCC)]

    def start(k):
        rc, cc = chunks[k]
        return pltpu.async_copy(
            out_hbm.at[pl.ds(base + rc * _CR, _CR), pl.ds(cc * _CC, _CC)],
            bufs[k % 2], sems[k % 2])

    rowsum_part = zeros
    pick_part = zeros
    g0_part = zeros
    t16 = tgt_v[...]
    pending = start(0)
    for k, (rc, cc) in enumerate(chunks):
        nxt = start(k + 1) if k + 1 < len(chunks) else None
        pending.wait()
        buf = bufs[k % 2]

        for r in range(_CR):
            tr = t16[rc * _CR + r]

            def body(i, acc):
                b = i * 256
                v = buf[r, pl.ds(b, _L)]
                for u in range(1, 16):
                    v = v + buf[r, pl.ds(b + u * _L, _L)]
                return acc + v
            acc = lax.fori_loop(0, _CC // 256, body, zeros)
            rowsum_part = rowsum_part + jnp.where(tr != PAD_, acc, zeros)

            oo = tr - cc * _CC
            inb = (oo >= 0) & (oo < _CC)
            win = jnp.where(inb, (oo // _L) * _L, 0)
            sel = jnp.where(inb & (tr != PAD_), oo % _L, -1)
            v = buf[r, pl.ds(win, _L)]
            pick_part = pick_part + jnp.where(lanes == sel, v, zeros)

            if cc == 0:
                v0 = buf[r, pl.ds(0, _L)]
                sel0 = jnp.where(tr != PAD_, 0, -1)
                g0_part = g0_part + jnp.where(lanes == sel0, v0, zeros)
        pending = nxt

    k_part = jnp.where(t16 != PAD_, K_, 0.0)
    part_v[...] = (k_part - SVAL_ * rowsum_part + SVAL_ * g0_part
                   - (CONF_ - SVAL_) * pick_part)
    pltpu.sync_copy(part_v, part_hbm.at[wid])


_sc_loss = functools.partial(
    pl.kernel,
    mesh=plsc.VectorSubcoreMesh(core_axis_name="c", subcore_axis_name="s"),
    out_type=jax.ShapeDtypeStruct((_NW, _L), jnp.float32),
    scratch_types=[
        pltpu.VMEM((_RPW,), jnp.int32),
        pltpu.VMEM((_CR, _CC), jnp.float32),
        pltpu.VMEM((_CR, _CC), jnp.float32),
        pltpu.VMEM((_L,), jnp.float32),
        pltpu.SemaphoreType.DMA,
        pltpu.SemaphoreType.DMA,
    ],
)(_sc_body)


@jax.jit
def kernel(output, target, one_hot):
    del one_hot
    t32 = target.astype(jnp.int32)
    sc_parts = _sc_loss(output, t32)

    t2 = t32.reshape(B_, 1)
    acc = pl.pallas_call(
        _tc_body,
        grid=(SIZE_ // BC_,),
        in_specs=[
            pl.BlockSpec((RTC_, BC_), lambda j: (0, j)),
            pl.BlockSpec((RTC_, 1), lambda j: (0, 0)),
        ],
        out_specs=pl.BlockSpec(
            (1, 1), lambda j: (0, 0), memory_space=pltpu.SMEM
        ),
        out_shape=jax.ShapeDtypeStruct((1, 1), jnp.float32),
    )(output, t2)
    return acc[0, 0] + jnp.sum(sc_parts)


# speedup vs baseline: 1.0006x; 1.0006x over previous
import math
import functools

import jax
import jax.numpy as jnp
from jax import lax
from jax.experimental import pallas as pl
from jax.experimental.pallas import tpu as pltpu
from jax.experimental.pallas import tpu_sc as plsc

SIZE_ = 32000
PAD_ = 0
SMOOTH_ = 0.1
CONF_ = 1.0 - SMOOTH_
SVAL_ = SMOOTH_ / (SIZE_ - 2)
K_ = (SIZE_ - 2) * SVAL_ * math.log(SVAL_) + CONF_ * math.log(CONF_)

B_ = 2048
RTC_ = 1536
BC_ = 1280

_NC = 2
_NS = 16
_NW = _NC * _NS
_RPW = (B_ - RTC_) // _NW
_L = 16
_CR = 8
_CC = 6400
_NRC = _RPW // _CR
_NCC = SIZE_ // _CC


def _tc_body(out_ref, t_ref, acc_ref):
    j = pl.program_id(0)
    out_blk = out_ref[...]
    t_blk = t_ref[...]
    nonpad = t_blk != PAD_

    colids = j * BC_ + lax.broadcasted_iota(jnp.int32, (RTC_, BC_), 1)
    w = jnp.where(colids == t_blk, CONF_, SVAL_)
    w = jnp.where(colids == 0, 0.0, w)
    w = jnp.where(nonpad, w, 0.0)
    partial = jnp.sum(out_blk * w)

    @pl.when(j == 0)
    def _init():
        cnt = jnp.sum(nonpad.astype(jnp.float32))
        acc_ref[0, 0] = K_ * cnt

    acc_ref[0, 0] = acc_ref[0, 0] - partial


def _sc_body(out_hbm, tgt_hbm, part_hbm, tgt_v, buf0, buf1, part_v,
             sem0, sem1):
    wid = lax.axis_index("s") * _NC + lax.axis_index("c")
    base = RTC_ + wid * _RPW
    pltpu.sync_copy(tgt_hbm.at[pl.ds(base, _RPW)], tgt_v)

    lanes = lax.iota(jnp.int32, _L)
    zeros = jnp.zeros((_L,), jnp.float32)

    bufs = (buf0, buf1)
    sems = (sem0, sem1)
    chunks = [(rc, cc) for rc in range(_NRC) for cc in range(_NCC)]

    def start(k):
        rc, cc = chunks[k]
        return pltpu.async_copy(
            out_hbm.at[pl.ds(base + rc * _CR, _CR), pl.ds(cc * _CC, _CC)],
            bufs[k % 2], sems[k % 2])

    rowsum_part = zeros
    pick_part = zeros
    g0_part = zeros
    t16 = tgt_v[...]
    pending = start(0)
    for k, (rc, cc) in enumerate(chunks):
        nxt = start(k + 1) if k + 1 < len(chunks) else None
        pending.wait()
        buf = bufs[k % 2]

        for r in range(_CR):
            tr = t16[rc * _CR + r]

            def body(i, acc):
                b = i * 256
                v = buf[r, pl.ds(b, _L)]
                for u in range(1, 16):
                    v = v + buf[r, pl.ds(b + u * _L, _L)]
                return acc + v
            acc = lax.fori_loop(0, _CC // 256, body, zeros)
            rowsum_part = rowsum_part + jnp.where(tr != PAD_, acc, zeros)

            oo = tr - cc * _CC
            inb = (oo >= 0) & (oo < _CC)
            win = jnp.where(inb, (oo // _L) * _L, 0)
            sel = jnp.where(inb & (tr != PAD_), oo % _L, -1)
            v = buf[r, pl.ds(win, _L)]
            pick_part = pick_part + jnp.where(lanes == sel, v, zeros)

            if cc == 0:
                v0 = buf[r, pl.ds(0, _L)]
                sel0 = jnp.where(tr != PAD_, 0, -1)
                g0_part = g0_part + jnp.where(lanes == sel0, v0, zeros)
        pending = nxt

    k_part = jnp.where(t16 != PAD_, K_, 0.0)
    part_v[...] = (k_part - SVAL_ * rowsum_part + SVAL_ * g0_part
                   - (CONF_ - SVAL_) * pick_part)
    pltpu.sync_copy(part_v, part_hbm.at[wid])


_sc_loss = functools.partial(
    pl.kernel,
    mesh=plsc.VectorSubcoreMesh(core_axis_name="c", subcore_axis_name="s"),
    out_type=jax.ShapeDtypeStruct((_NW, _L), jnp.float32),
    scratch_types=[
        pltpu.VMEM((_RPW,), jnp.int32),
        pltpu.VMEM((_CR, _CC), jnp.float32),
        pltpu.VMEM((_CR, _CC), jnp.float32),
        pltpu.VMEM((_L,), jnp.float32),
        pltpu.SemaphoreType.DMA,
        pltpu.SemaphoreType.DMA,
    ],
)(_sc_body)


@jax.jit
def kernel(output, target, one_hot):
    del one_hot
    t32 = target.astype(jnp.int32)
    t2 = t32.reshape(B_, 1)
    acc = pl.pallas_call(
        _tc_body,
        grid=(SIZE_ // BC_,),
        in_specs=[
            pl.BlockSpec((RTC_, BC_), lambda j: (0, j)),
            pl.BlockSpec((RTC_, 1), lambda j: (0, 0)),
        ],
        out_specs=pl.BlockSpec(
            (1, 1), lambda j: (0, 0), memory_space=pltpu.SMEM
        ),
        out_shape=jax.ShapeDtypeStruct((1, 1), jnp.float32),
    )(output, t2)
    sc_parts = _sc_loss(output, t32)
    return acc[0, 0] + jnp.sum(sc_parts)


# speedup vs baseline: 1.2561x; 1.2554x over previous
import math
import functools

import jax
import jax.numpy as jnp
from jax import lax
from jax.experimental import pallas as pl
from jax.experimental.pallas import tpu as pltpu

SIZE_ = 32000
PAD_ = 0
SMOOTH_ = 0.1
CONF_ = 1.0 - SMOOTH_
SVAL_ = SMOOTH_ / (SIZE_ - 2)
K_ = (SIZE_ - 2) * SVAL_ * math.log(SVAL_) + CONF_ * math.log(CONF_)

B_ = 2048
BC_ = 3200


def _loss_body(out_ref, t_ref, acc_ref):
    j = pl.program_id(0)
    out_blk = out_ref[...]
    t_blk = t_ref[...]
    nonpad = t_blk != PAD_

    col0 = j * BC_
    colids = col0 + lax.broadcasted_iota(jnp.int32, (B_, BC_), 1)
    w = jnp.where(colids == t_blk, CONF_, SVAL_)
    w = jnp.where(colids == 0, 0.0, w)
    w = jnp.where(nonpad, w, 0.0)
    partial = jnp.sum(out_blk * w)

    @pl.when(j == 0)
    def _init():
        cnt = jnp.sum(nonpad.astype(jnp.float32))
        acc_ref[0, 0] = K_ * cnt

    acc_ref[0, 0] = acc_ref[0, 0] - partial


@jax.jit
def kernel(output, target, one_hot):
    del one_hot
    t2 = target.astype(jnp.int32).reshape(B_, 1)
    acc = pl.pallas_call(
        _loss_body,
        grid=(SIZE_ // BC_,),
        in_specs=[
            pl.BlockSpec((B_, BC_), lambda j: (0, j)),
            pl.BlockSpec((B_, 1), lambda j: (0, 0)),
        ],
        out_specs=pl.BlockSpec(
            (1, 1), lambda j: (0, 0), memory_space=pltpu.SMEM
        ),
        out_shape=jax.ShapeDtypeStruct((1, 1), jnp.float32),
    )(output, t2)
    return acc[0, 0]


# speedup vs baseline: 1.2720x; 1.0126x over previous
import math

import jax
import jax.numpy as jnp
from jax import lax
from jax.experimental import pallas as pl
from jax.experimental.pallas import tpu as pltpu

SIZE_ = 32000
PAD_ = 0
SMOOTH_ = 0.1
CONF_ = 1.0 - SMOOTH_
SVAL_ = SMOOTH_ / (SIZE_ - 2)
K_ = (SIZE_ - 2) * SVAL_ * math.log(SVAL_) + CONF_ * math.log(CONF_)

B_ = 2048
BR_ = 128


def _loss_body(out_ref, t_ref, acc_ref):
    j = pl.program_id(0)
    out_blk = out_ref[...]
    t_blk = t_ref[...]
    nonpad = t_blk != PAD_

    colids = lax.broadcasted_iota(jnp.int32, (BR_, SIZE_), 1)
    w = jnp.where(colids == t_blk, CONF_, SVAL_)
    w = jnp.where(colids == 0, 0.0, w)
    w = jnp.where(nonpad, w, 0.0)
    partial = jnp.sum(out_blk * w)
    cnt = jnp.sum(nonpad.astype(jnp.float32))

    @pl.when(j == 0)
    def _init():
        acc_ref[0, 0] = 0.0

    acc_ref[0, 0] = acc_ref[0, 0] + (K_ * cnt - partial)


@jax.jit
def kernel(output, target, one_hot):
    del one_hot
    t2 = target.astype(jnp.int32).reshape(B_, 1)
    acc = pl.pallas_call(
        _loss_body,
        grid=(B_ // BR_,),
        in_specs=[
            pl.BlockSpec((BR_, SIZE_), lambda j: (j, 0)),
            pl.BlockSpec((BR_, 1), lambda j: (j, 0)),
        ],
        out_specs=pl.BlockSpec(
            (1, 1), lambda j: (0, 0), memory_space=pltpu.SMEM
        ),
        out_shape=jax.ShapeDtypeStruct((1, 1), jnp.float32),
    )(output, t2)
    return acc[0, 0]


# speedup vs baseline: 1.3728x; 1.0793x over previous
import math

import jax
import jax.numpy as jnp
from jax import lax
from jax.experimental import pallas as pl
from jax.experimental.pallas import tpu as pltpu

SIZE_ = 32000
PAD_ = 0
SMOOTH_ = 0.1
CONF_ = 1.0 - SMOOTH_
SVAL_ = SMOOTH_ / (SIZE_ - 2)
K_ = (SIZE_ - 2) * SVAL_ * math.log(SVAL_) + CONF_ * math.log(CONF_)

B_ = 2048
BR_ = 128


def _loss_body(out_ref, t_ref, acc_ref):
    j = pl.program_id(0)
    out_blk = out_ref[...]
    t_blk = t_ref[...]
    nonpad = t_blk != PAD_

    colids = lax.broadcasted_iota(jnp.int32, (BR_, SIZE_), 1)
    rowsum = jnp.sum(out_blk, axis=1, keepdims=True)
    pick = jnp.sum(jnp.where(colids == t_blk, out_blk, 0.0),
                   axis=1, keepdims=True)
    out0 = out_blk[:, 0:1]
    per_row = K_ - SVAL_ * (rowsum - out0) - (CONF_ - SVAL_) * pick
    partial = jnp.sum(jnp.where(nonpad, per_row, 0.0))

    @pl.when(j == 0)
    def _init():
        acc_ref[0, 0] = 0.0

    acc_ref[0, 0] = acc_ref[0, 0] + partial


@jax.jit
def kernel(output, target, one_hot):
    del one_hot
    t2 = target.astype(jnp.int32).reshape(B_, 1)
    acc = pl.pallas_call(
        _loss_body,
        grid=(B_ // BR_,),
        in_specs=[
            pl.BlockSpec((BR_, SIZE_), lambda j: (j, 0)),
            pl.BlockSpec((BR_, 1), lambda j: (j, 0)),
        ],
        out_specs=pl.BlockSpec(
            (1, 1), lambda j: (0, 0), memory_space=pltpu.SMEM
        ),
        out_shape=jax.ShapeDtypeStruct((1, 1), jnp.float32),
    )(output, t2)
    return acc[0, 0]
